# Initial kernel scaffold; baseline (speedup 1.0000x reference)
import jax
import jax.numpy as jnp
from jax.experimental import pallas as pl


def kernel(input_x, table):
    raise NotImplementedError("write your pallas kernel here")


# speedup vs baseline: 3.6836x; 3.6836x over previous
import functools

import jax
import jax.numpy as jnp
from jax import lax
from jax.experimental import pallas as pl
from jax.experimental.pallas import tpu as pltpu
from jax.experimental.pallas import tpu_sc as plsc

EMBED_DIM = 64
NUM_CORES = 2
NUM_SUBCORES = 16
NUM_WORKERS = NUM_CORES * NUM_SUBCORES
CHUNK = 1024


@functools.cache
def _make_gather(num_rows: int):
    assert num_rows % (NUM_WORKERS * CHUNK) == 0
    rows_per_worker = num_rows // NUM_WORKERS
    n_chunks = rows_per_worker // CHUNK
    mesh = plsc.VectorSubcoreMesh(core_axis_name="c", subcore_axis_name="s")

    @functools.partial(
        pl.kernel,
        mesh=mesh,
        compiler_params=pltpu.CompilerParams(use_tc_tiling_on_sc=False),
        out_type=jax.ShapeDtypeStruct((num_rows, EMBED_DIM), jnp.float32),
        scratch_types=[
            pltpu.VMEM((CHUNK,), jnp.int32),
            pltpu.VMEM((CHUNK, EMBED_DIM), jnp.float32),
            pltpu.SemaphoreType.DMA,
        ],
    )
    def gather_kernel(idx_hbm, table_hbm, out_hbm, idx_v, rows_v, sem):
        wid = lax.axis_index("s") * NUM_CORES + lax.axis_index("c")
        base = wid * rows_per_worker

        def step(i, carry):
            off = base + i * CHUNK
            pltpu.sync_copy(idx_hbm.at[pl.ds(off, CHUNK)], idx_v)
            pltpu.async_copy(table_hbm.at[idx_v], rows_v, sem).wait()
            pltpu.sync_copy(rows_v, out_hbm.at[pl.ds(off, CHUNK)])
            return carry

        lax.fori_loop(0, n_chunks, step, 0)

    return gather_kernel


def kernel(input_x, table):
    batch, hist = input_x.shape
    idx = input_x.reshape(-1).astype(jnp.int32)
    y = _make_gather(idx.shape[0])(idx, table)
    y = y.reshape(batch, hist, EMBED_DIM)
    return (y, y)


# speedup vs baseline: 3.7658x; 1.0223x over previous
import functools

import jax
import jax.numpy as jnp
from jax import lax
from jax.experimental import pallas as pl
from jax.experimental.pallas import tpu as pltpu
from jax.experimental.pallas import tpu_sc as plsc

EMBED_DIM = 64
NUM_CORES = 2
NUM_SUBCORES = 16
NUM_WORKERS = NUM_CORES * NUM_SUBCORES
CHUNK = 800
NBUF = 2


@functools.cache
def _make_gather(num_rows: int):
    assert num_rows % (NUM_WORKERS * CHUNK) == 0
    rows_per_worker = num_rows // NUM_WORKERS
    n_chunks = rows_per_worker // CHUNK
    mesh = plsc.VectorSubcoreMesh(core_axis_name="c", subcore_axis_name="s")

    @functools.partial(
        pl.kernel,
        mesh=mesh,
        compiler_params=pltpu.CompilerParams(use_tc_tiling_on_sc=False),
        out_type=jax.ShapeDtypeStruct((num_rows, EMBED_DIM), jnp.float32),
        scratch_types=[
            pltpu.VMEM((rows_per_worker,), jnp.int32),
            pltpu.VMEM((NBUF, CHUNK, EMBED_DIM), jnp.float32),
            pltpu.SemaphoreType.DMA((NBUF,)),
            pltpu.SemaphoreType.DMA((NBUF,)),
        ],
    )
    def gather_kernel(idx_hbm, table_hbm, out_hbm, idx_v, rows_v, gsem, osem):
        wid = lax.axis_index("s") * NUM_CORES + lax.axis_index("c")
        base = wid * rows_per_worker
        pltpu.sync_copy(idx_hbm.at[pl.ds(base, rows_per_worker)], idx_v)

        def gather_start(i, slot):
            pltpu.async_copy(
                table_hbm.at[idx_v.at[pl.ds(i * CHUNK, CHUNK)]],
                rows_v.at[slot],
                gsem.at[slot],
            )

        def gather_wait(slot):
            pltpu.make_async_copy(
                table_hbm.at[idx_v.at[pl.ds(0, CHUNK)]],
                rows_v.at[slot],
                gsem.at[slot],
            ).wait()

        def out_start(i, slot):
            pltpu.async_copy(
                rows_v.at[slot],
                out_hbm.at[pl.ds(base + i * CHUNK, CHUNK)],
                osem.at[slot],
            )

        def out_wait(slot):
            pltpu.make_async_copy(
                rows_v.at[slot],
                out_hbm.at[pl.ds(base, CHUNK)],
                osem.at[slot],
            ).wait()

        gather_start(0, 0)

        def step(i, carry):
            slot = lax.rem(i, NBUF)
            nxt = lax.rem(i + 1, NBUF)

            @pl.when(i + 1 < n_chunks)
            def _():
                @pl.when(i + 1 >= NBUF)
                def _():
                    out_wait(nxt)

                gather_start(i + 1, nxt)

            gather_wait(slot)
            out_start(i, slot)
            return carry

        lax.fori_loop(0, n_chunks, step, 0)
        for s in range(NBUF):
            out_wait(s)

    return gather_kernel


def kernel(input_x, table):
    batch, hist = input_x.shape
    idx = input_x.reshape(-1).astype(jnp.int32)
    y = _make_gather(idx.shape[0])(idx, table)
    y = y.reshape(batch, hist, EMBED_DIM)
    return (y, y)
